# baseline (speedup 1.0000x reference)
import dataclasses
import functools

import jax
import jax.numpy as jnp
from jax import lax
from jax.experimental import pallas as pl
from jax.experimental.pallas import tpu as pltpu
from jax.experimental.pallas import tpu_sc as plsc

T = 1025
TA = 513
TB = 512
C = 96
DPAD = 528

BB = 4


def _match_body(x_ref, dst_ref, scale_ref):
    for bb in range(BB):
        A = x_ref[bb, pl.Slice(0, TA, 2), :]
        B = x_ref[bb, pl.Slice(1, TB, 2), :]
        na = A / jnp.sqrt(jnp.sum(A * A, axis=-1, keepdims=True))
        nb = B / jnp.sqrt(jnp.sum(B * B, axis=-1, keepdims=True))
        scores = jax.lax.dot_general(
            na, nb, (((1,), (1,)), ((), ())), preferred_element_type=jnp.float32
        )
        mx = jnp.max(scores, axis=-1, keepdims=True)
        lane = jax.lax.broadcasted_iota(jnp.int32, (TA, TB), 1)
        dst = jnp.min(jnp.where(scores == mx, lane, TB), axis=-1, keepdims=True)
        row = jax.lax.broadcasted_iota(jnp.int32, (TA, 1), 0)
        dstm = jnp.where(row == 0, -1, dst)
        P = (lane == dstm).astype(jnp.float32)
        counts = jnp.sum(P, axis=0, keepdims=True)
        scale_ref[bb] = 1.0 / (1.0 + counts)
        pad = jnp.zeros((DPAD - TA, 1), jnp.int32)
        dst_ref[bb] = jnp.concatenate([dst, pad], axis=0)


@jax.jit
def _match(x):
    nb_ = x.shape[0]
    return pl.pallas_call(
        _match_body,
        grid=(nb_ // BB,),
        in_specs=[pl.BlockSpec((BB, T, C), lambda i: (i, 0, 0))],
        out_specs=[
            pl.BlockSpec((BB, DPAD, 1), lambda i: (i, 0, 0)),
            pl.BlockSpec((BB, 1, TB), lambda i: (i, 0, 0)),
        ],
        out_shape=[
            jax.ShapeDtypeStruct((nb_, DPAD, 1), jnp.int32),
            jax.ShapeDtypeStruct((nb_, 1, TB), jnp.float32),
        ],
    )(x)


NW = 32
V = 16
CHUNK = 128


def _merge_sc(x, dstt, scalet):
    nb_ = x.shape[0]
    bpw = nb_ // NW
    mesh = plsc.VectorSubcoreMesh(core_axis_name="c", subcore_axis_name="s")
    cp = pltpu.CompilerParams()
    if "needs_layout_passes" in pltpu.CompilerParams.__dataclass_fields__:
        cp = dataclasses.replace(cp, needs_layout_passes=False)

    @functools.partial(
        pl.kernel,
        mesh=mesh,
        compiler_params=cp,
        out_type=jax.ShapeDtypeStruct((nb_, TA, C), jnp.float32),
        scratch_types=[
            pltpu.VMEM((CHUNK, C), jnp.float32),
            pltpu.VMEM((TA, C), jnp.float32),
            pltpu.VMEM((DPAD,), jnp.int32),
            pltpu.VMEM((TB,), jnp.float32),
            pltpu.SemaphoreType.DMA,
        ],
    )
    def k(x_hbm, dst_hbm, scl_hbm, out_hbm, buf, acc, dv, sv, sem):
        wid = lax.axis_index("s") * 2 + lax.axis_index("c")
        iota = lax.broadcasted_iota(jnp.int32, (V,), 0)

        @pl.loop(0, bpw)
        def _batch(bslot):
            bi = wid * bpw + bslot
            pltpu.async_copy(dst_hbm.at[bi], dv, sem).wait()
            pltpu.async_copy(scl_hbm.at[bi], sv, sem).wait()

            @pl.loop(1, TA)
            def _z(j):
                for k6 in range(6):
                    acc[j, pl.ds(k6 * V, V)] = jnp.zeros((V,), jnp.float32)

            @pl.loop(0, T // CHUNK)
            def _chunk(c):
                pltpu.async_copy(
                    x_hbm.at[bi, pl.ds(c * CHUNK, CHUNK), :], buf, sem
                ).wait()

                @pl.when(c == 0)
                def _cls():
                    for k6 in range(6):
                        acc[0, pl.ds(k6 * V, V)] = buf[0, pl.ds(k6 * V, V)]

                @pl.loop(0, 4)
                def _grp(g):
                    i0 = c * 64 + g * 16
                    dvec = dv[pl.ds(i0, V)]
                    srow = dvec + 1
                    amask = (iota + i0) > 0
                    arows = iota * 2 + g * 32
                    brows = arows + 1
                    trow = iota + (i0 + 1)
                    for cc in range(C):
                        ccv = jnp.full((V,), cc, jnp.int32)
                        av = plsc.load_gather(buf, [arows, ccv])
                        plsc.addupdate_scatter(acc, [srow, ccv], av, mask=amask)
                        bv = plsc.load_gather(buf, [brows, ccv])
                        plsc.addupdate_scatter(acc, [trow, ccv], bv)

            pltpu.async_copy(
                x_hbm.at[bi, pl.ds(1024, 1), :], buf.at[pl.ds(0, 1), :], sem
            ).wait()
            svec = plsc.load_gather(dv, [jnp.full((V,), TB, jnp.int32)]) + 1
            for k6 in range(6):
                vals = buf[0, pl.ds(k6 * V, V)]
                plsc.addupdate_scatter(acc, [svec, k6 * V + iota], vals)

            @pl.loop(0, TB // V)
            def _d(q):
                s16 = sv[pl.ds(q * V, V)]
                rr = iota + (q * V + 1)
                for cc in range(C):
                    ccv = jnp.full((V,), cc, jnp.int32)
                    vals = plsc.load_gather(acc, [rr, ccv]) * s16
                    plsc.store_scatter(acc, [rr, ccv], vals)

            pltpu.async_copy(acc, out_hbm.at[bi], sem).wait()

    return k(x, dstt, scalet)


@jax.jit
def kernel(hidden_states):
    dstt, scalet = _match(hidden_states)
    dst1 = dstt[:, :, 0]
    scl1 = scalet[:, 0, :]
    return _merge_sc(hidden_states, dst1, scl1)


# speedup vs baseline: 4.7962x; 4.7962x over previous
import functools

import jax
import jax.numpy as jnp
from jax.experimental import pallas as pl

T = 1025
TA = 513
TB = 512
C = 96


BB = 8


def _tome_body(x_ref, out_ref):
    for bb in range(BB):
        _tome_one(x_ref, out_ref, bb)


def _tome_one(x_ref, out_ref, bb):
    A = x_ref[bb, pl.Slice(0, TA, 2), :]
    B = x_ref[bb, pl.Slice(1, TB, 2), :]
    na = A / jnp.sqrt(jnp.sum(A * A, axis=-1, keepdims=True))
    nb = B / jnp.sqrt(jnp.sum(B * B, axis=-1, keepdims=True))
    scores = jax.lax.dot_general(
        na, nb, (((1,), (1,)), ((), ())), preferred_element_type=jnp.float32
    )
    mx = jnp.max(scores, axis=-1, keepdims=True)
    lane = jax.lax.broadcasted_iota(jnp.int32, (TA, TB), 1)
    dst = jnp.min(jnp.where(scores == mx, lane, TB), axis=-1, keepdims=True)
    row = jax.lax.broadcasted_iota(jnp.int32, (TA, 1), 0)
    dst = jnp.where(row == 0, -1, dst)
    P = (lane == dst).astype(jnp.float32)
    A1 = jnp.concatenate([A, jnp.ones((TA, 1), jnp.float32)], axis=1)
    M = jax.lax.dot_general(
        P, A1, (((0,), (0,)), ((), ())), preferred_element_type=jnp.float32
    )
    merged = M[:, :C]
    counts = M[:, C : C + 1]
    dst_rows = (B + merged) / (1.0 + counts)
    out_ref[bb] = jnp.concatenate([A[0:1, :], dst_rows], axis=0)


@functools.partial(jax.jit, static_argnames=("interpret",))
def _tome(x, interpret=False):
    nb_ = x.shape[0]
    return pl.pallas_call(
        _tome_body,
        grid=(nb_ // BB,),
        in_specs=[
            pl.BlockSpec((BB, T, C), lambda i: (i, 0, 0)),
        ],
        out_specs=pl.BlockSpec((BB, TA, C), lambda i: (i, 0, 0)),
        out_shape=jax.ShapeDtypeStruct((nb_, TA, C), jnp.float32),
        interpret=interpret,
    )(x)


def kernel(hidden_states):
    return _tome(hidden_states)
